# baseline (speedup 1.0000x reference)
import jax
import jax.numpy as jnp
from jax import lax
from jax.experimental import pallas as pl
from jax.experimental.pallas import tpu as pltpu
from jax.experimental.pallas import tpu_sc as plsc

T = 32
DIM = 1024
INTER = 2048
E = 16
TOPK = 2
IB = 1024
NJ = INTER // IB
NP = T * TOPK


def _scale_sc_kernel(pk_hbm, eo_hbm, s_hbm, pk_v, s_v):
    del eo_hbm
    cid = lax.axis_index("c")
    sid = lax.axis_index("s")

    @pl.when((cid == 0) & (sid == 0))
    def _():
        pltpu.sync_copy(pk_hbm, pk_v)

        for z in range(E * T // 16):
            s_v[pl.ds(z * 16, 16)] = jnp.zeros((16,), jnp.float32)
        for c in range(NP // 16):
            p = c * 16 + lax.iota(jnp.int32, 16)
            t = lax.shift_right_logical(p, 1)
            idx = pk_v[pl.ds(c * 16, 16)]
            lin = idx * T + t
            gw = plsc.bitcast(pk_v[pl.ds(NP + c * 16, 16)], jnp.float32)
            plsc.addupdate_scatter(s_v, [lin], gw)
        pltpu.sync_copy(s_v, s_hbm)


def _routing_scales(expert_indices, expert_weights, eo):
    gw_bits = jax.lax.bitcast_convert_type(expert_weights, jnp.int32).reshape(NP)
    pk = jnp.concatenate([expert_indices.astype(jnp.int32).reshape(NP), gw_bits])
    mesh = plsc.VectorSubcoreMesh(
        core_axis_name="c", subcore_axis_name="s", num_cores=1
    )
    s = pl.kernel(
        _scale_sc_kernel,
        mesh=mesh,
        out_type=jax.ShapeDtypeStruct((E * T,), jnp.float32),
        scratch_types=[
            pltpu.VMEM((2 * NP,), jnp.int32),
            pltpu.VMEM((E * T,), jnp.float32),
        ],
        compiler_params=pltpu.CompilerParams(needs_layout_passes=False),
    )(pk, eo)
    return s.reshape(E, T, 1)


def _ffn_kernel(x_ref, w1_ref, w3_ref, w2_ref, eo_ref):
    j = pl.program_id(1)

    @pl.when(j == 0)
    def _init():
        eo_ref[...] = jnp.zeros_like(eo_ref)

    x = x_ref[...]
    w1 = w1_ref[0]
    w3 = w3_ref[0]
    w2 = w2_ref[0]

    dn = (((1,), (1,)), ((), ()))
    x1 = jax.lax.dot_general(x, w1, dn, preferred_element_type=jnp.float32)
    x3 = jax.lax.dot_general(x, w3, dn, preferred_element_type=jnp.float32)
    h = x1 * jax.nn.sigmoid(x1) * x3
    eo_ref[0] += jax.lax.dot_general(h, w2, dn, preferred_element_type=jnp.float32)


def _combine_kernel(s_ref, eo_ref, out_ref):
    out_ref[...] = jnp.sum(s_ref[...] * eo_ref[...], axis=0)


def kernel(x, expert_indices, expert_weights, w1, w2, w3):
    eo = pl.pallas_call(
        _ffn_kernel,
        grid=(E, NJ),
        in_specs=[
            pl.BlockSpec((T, DIM), lambda e, j: (0, 0)),
            pl.BlockSpec((1, IB, DIM), lambda e, j: (e, j, 0)),
            pl.BlockSpec((1, IB, DIM), lambda e, j: (e, j, 0)),
            pl.BlockSpec((1, DIM, IB), lambda e, j: (e, 0, j)),
        ],
        out_specs=pl.BlockSpec((1, T, DIM), lambda e, j: (e, 0, 0)),
        out_shape=jax.ShapeDtypeStruct((E, T, DIM), jnp.float32),
    )(x, w1, w3, w2)
    s = _routing_scales(expert_indices, expert_weights, eo)
    return pl.pallas_call(
        _combine_kernel,
        out_shape=jax.ShapeDtypeStruct((T, DIM), jnp.float32),
    )(s, eo)


# speedup vs baseline: 1.0129x; 1.0129x over previous
import jax
import jax.numpy as jnp
from jax import lax
from jax.experimental import pallas as pl
from jax.experimental.pallas import tpu as pltpu
from jax.experimental.pallas import tpu_sc as plsc

T = 32
DIM = 1024
INTER = 2048
E = 16
TOPK = 2
IB = 1024
NJ = INTER // IB
NP = T * TOPK


def _scale_sc_kernel(pk_hbm, s_hbm, pk_v, s_v):
    cid = lax.axis_index("c")
    sid = lax.axis_index("s")

    @pl.when((cid == 0) & (sid == 0))
    def _():
        pltpu.sync_copy(pk_hbm, pk_v)

        for z in range(E * T // 16):
            s_v[pl.ds(z * 16, 16)] = jnp.zeros((16,), jnp.float32)
        for c in range(NP // 16):
            p = c * 16 + lax.iota(jnp.int32, 16)
            t = lax.shift_right_logical(p, 1)
            idx = pk_v[pl.ds(c * 16, 16)]
            lin = idx * T + t
            gw = plsc.bitcast(pk_v[pl.ds(NP + c * 16, 16)], jnp.float32)
            plsc.addupdate_scatter(s_v, [lin], gw)
        pltpu.sync_copy(s_v, s_hbm)


def _routing_scales(expert_indices, expert_weights):
    gw_bits = jax.lax.bitcast_convert_type(expert_weights, jnp.int32).reshape(NP)
    pk = jnp.concatenate([expert_indices.astype(jnp.int32).reshape(NP), gw_bits])
    mesh = plsc.VectorSubcoreMesh(
        core_axis_name="c", subcore_axis_name="s", num_cores=1
    )
    s = pl.kernel(
        _scale_sc_kernel,
        mesh=mesh,
        out_type=jax.ShapeDtypeStruct((E * T,), jnp.float32),
        scratch_types=[
            pltpu.VMEM((2 * NP,), jnp.int32),
            pltpu.VMEM((E * T,), jnp.float32),
        ],
        compiler_params=pltpu.CompilerParams(needs_layout_passes=False),
    )(pk)
    return s.reshape(E, T, 1)


def _ffn_kernel(x_ref, w1_ref, w3_ref, w2_ref, eo_ref):
    j = pl.program_id(1)

    @pl.when(j == 0)
    def _init():
        eo_ref[...] = jnp.zeros_like(eo_ref)

    x = x_ref[...]
    w1 = w1_ref[0]
    w3 = w3_ref[0]
    w2 = w2_ref[0]

    dn = (((1,), (1,)), ((), ()))
    x1 = jax.lax.dot_general(x, w1, dn, preferred_element_type=jnp.float32)
    x3 = jax.lax.dot_general(x, w3, dn, preferred_element_type=jnp.float32)
    h = x1 * jax.nn.sigmoid(x1) * x3
    eo_ref[0] += jax.lax.dot_general(h, w2, dn, preferred_element_type=jnp.float32)


def _combine_kernel(s_ref, eo_ref, out_ref):
    out_ref[...] = jnp.sum(s_ref[...] * eo_ref[...], axis=0)


def kernel(x, expert_indices, expert_weights, w1, w2, w3):
    eo = pl.pallas_call(
        _ffn_kernel,
        grid=(E, NJ),
        in_specs=[
            pl.BlockSpec((T, DIM), lambda e, j: (0, 0)),
            pl.BlockSpec((1, IB, DIM), lambda e, j: (e, j, 0)),
            pl.BlockSpec((1, IB, DIM), lambda e, j: (e, j, 0)),
            pl.BlockSpec((1, DIM, IB), lambda e, j: (e, 0, j)),
        ],
        out_specs=pl.BlockSpec((1, T, DIM), lambda e, j: (e, 0, 0)),
        out_shape=jax.ShapeDtypeStruct((E, T, DIM), jnp.float32),
    )(x, w1, w3, w2)
    s = _routing_scales(expert_indices, expert_weights)
    return pl.pallas_call(
        _combine_kernel,
        out_shape=jax.ShapeDtypeStruct((T, DIM), jnp.float32),
    )(s, eo)
